# baseline (speedup 1.0000x reference)
import functools

import jax
import jax.numpy as jnp
from jax import lax
from jax.experimental import pallas as pl
from jax.experimental.pallas import tpu as pltpu
from jax.experimental.pallas import tpu_sc as plsc

IN_CH = 128
OUT_CH = 128
N_NODES = 10000
N_EDGES = 320000

CHUNK = 128
N_CHUNKS = N_EDGES // CHUNK
NC = 2
NS = 16
NW = NC * NS
SL0_CHUNKS = 2000
SL1_CHUNKS = 500
CPW0 = 64
CPW1 = 16
NBUF = 5
DA = 2
DW = 4

NODE_BLK = 2000


def _tables_body(x_ref, w1_ref, eps_ref, a_ref, b_ref):
    xw = jnp.dot(x_ref[...], w1_ref[...], preferred_element_type=jnp.float32)
    b_ref[...] = xw
    a_ref[...] = (1.0 + eps_ref[0]) * xw


def _make_tables(x, W1, eps):
    grid = (N_NODES // NODE_BLK,)
    return pl.pallas_call(
        _tables_body,
        grid=grid,
        in_specs=[
            pl.BlockSpec((NODE_BLK, IN_CH), lambda i: (i, 0)),
            pl.BlockSpec((IN_CH, OUT_CH), lambda i: (0, 0)),
            pl.BlockSpec(memory_space=pltpu.SMEM),
        ],
        out_specs=[
            pl.BlockSpec((NODE_BLK, OUT_CH), lambda i: (i, 0)),
            pl.BlockSpec((NODE_BLK, OUT_CH), lambda i: (i, 0)),
        ],
        out_shape=[
            jax.ShapeDtypeStruct((N_NODES, OUT_CH), jnp.float32),
            jax.ShapeDtypeStruct((N_NODES, OUT_CH), jnp.float32),
        ],
    )(x, W1, eps)


def _sc_gather_body_for(sl_chunks, ch_per_w):
    def body(a_hbm, b_hbm, row_hbm, col_hbm, g_hbm,
             idx_r, idx_c, rows0, rows1, rows2, rows3, rows4,
             sa0, sa1, sa2, sa3, sa4, sb0, sb1, sb2, sb3, sb4):
        wid = lax.axis_index("s") * NC + lax.axis_index("c")
        base_chunk = wid * ch_per_w
        rows = (rows0, rows1, rows2, rows3, rows4)
        sem_a = (sa0, sa1, sa2, sa3, sa4)
        sem_b = (sb0, sb1, sb2, sb3, sb4)

        pltpu.sync_copy(row_hbm.at[pl.ds(base_chunk, ch_per_w)], idx_r)
        pltpu.sync_copy(col_hbm.at[pl.ds(base_chunk, ch_per_w)], idx_c)

        def valid(k):
            return (k >= 0) & (k < ch_per_w) & (base_chunk + k < sl_chunks)

        @pl.loop(0, ch_per_w + NBUF, step=NBUF)
        def _(k0):
            for b in range(NBUF):
                k = k0 + b

                @pl.when(valid(k))
                def _():
                    pltpu.async_copy(
                        a_hbm.at[idx_r.at[k]], rows[b], sem_a[b]
                    )

                ba = (b - DA) % NBUF
                ka = k - DA

                @pl.when(valid(ka))
                def _():
                    pltpu.make_async_copy(
                        a_hbm.at[idx_r.at[ka]], rows[ba], sem_a[ba]
                    ).wait()
                    pltpu.async_copy(
                        b_hbm.at[idx_c.at[ka]], rows[ba], sem_b[ba],
                        add=True,
                    )

                bw = (b - DW) % NBUF
                kw = k - DW

                @pl.when(valid(kw))
                def _():
                    pltpu.make_async_copy(
                        b_hbm.at[idx_c.at[kw]], rows[bw], sem_b[bw]
                    ).wait()
                    pltpu.sync_copy(
                        rows[bw],
                        g_hbm.at[pl.ds((base_chunk + kw) * CHUNK, CHUNK)],
                    )

    return body


@functools.cache
def _make_sc_gather(sl_chunks, ch_per_w):
    mesh = plsc.VectorSubcoreMesh(core_axis_name="c", subcore_axis_name="s")
    return pl.kernel(
        _sc_gather_body_for(sl_chunks, ch_per_w),
        out_type=jax.ShapeDtypeStruct(
            (sl_chunks * CHUNK, OUT_CH), jnp.float32
        ),
        mesh=mesh,
        scratch_types=[
            pltpu.VMEM((ch_per_w, CHUNK), jnp.int32),
            pltpu.VMEM((ch_per_w, CHUNK), jnp.int32),
            pltpu.VMEM((CHUNK, OUT_CH), jnp.float32),
            pltpu.VMEM((CHUNK, OUT_CH), jnp.float32),
            pltpu.VMEM((CHUNK, OUT_CH), jnp.float32),
            pltpu.VMEM((CHUNK, OUT_CH), jnp.float32),
            pltpu.VMEM((CHUNK, OUT_CH), jnp.float32),
            pltpu.SemaphoreType.DMA,
            pltpu.SemaphoreType.DMA,
            pltpu.SemaphoreType.DMA,
            pltpu.SemaphoreType.DMA,
            pltpu.SemaphoreType.DMA,
            pltpu.SemaphoreType.DMA,
            pltpu.SemaphoreType.DMA,
            pltpu.SemaphoreType.DMA,
            pltpu.SemaphoreType.DMA,
            pltpu.SemaphoreType.DMA,
        ],
    )


EDGE_BLK = 8000


def _mlp_body(g_ref, b1_ref, w2_ref, b2_ref, out_ref):
    h = jnp.maximum(g_ref[...] + b1_ref[...], 0.0)
    out_ref[...] = (
        jnp.dot(h, w2_ref[...], preferred_element_type=jnp.float32)
        + b2_ref[...]
    )


def _mlp_body_acc(acc_ref, g_ref, b1_ref, w2_ref, b2_ref, out_ref):
    del acc_ref
    _mlp_body(g_ref, b1_ref, w2_ref, b2_ref, out_ref)


def _mlp_part(acc, g, b1r, W2, b2r, blk_off):
    n_part = g.shape[0]
    common_in = [
        pl.BlockSpec((EDGE_BLK, OUT_CH), lambda i: (i, 0)),
        pl.BlockSpec((1, OUT_CH), lambda i: (0, 0)),
        pl.BlockSpec((OUT_CH, OUT_CH), lambda i: (0, 0)),
        pl.BlockSpec((1, OUT_CH), lambda i: (0, 0)),
    ]
    out_spec = pl.BlockSpec(
        (EDGE_BLK, OUT_CH), lambda i, o=blk_off: (i + o, 0)
    )
    out_shape = jax.ShapeDtypeStruct((N_EDGES, OUT_CH), jnp.float32)
    if acc is None:
        return pl.pallas_call(
            _mlp_body,
            grid=(n_part // EDGE_BLK,),
            in_specs=common_in,
            out_specs=out_spec,
            out_shape=out_shape,
        )(g, b1r, W2, b2r)
    return pl.pallas_call(
        _mlp_body_acc,
        grid=(n_part // EDGE_BLK,),
        in_specs=[pl.BlockSpec(memory_space=pl.ANY)] + common_in,
        out_specs=out_spec,
        out_shape=out_shape,
        input_output_aliases={0: 0},
    )(acc, g, b1r, W2, b2r)


def kernel(x, edge_index, W1, b1, W2, b2, eps):
    idx2d = edge_index.astype(jnp.int32).reshape(2, N_CHUNKS, CHUNK)
    idx0 = jnp.pad(
        idx2d[:, :SL0_CHUNKS],
        ((0, 0), (0, CPW0 * NW - SL0_CHUNKS), (0, 0)),
    )
    idx1 = jnp.pad(
        idx2d[:, SL0_CHUNKS:],
        ((0, 0), (0, CPW1 * NW - SL1_CHUNKS), (0, 0)),
    )
    a_tab, b_tab = _make_tables(x, W1, eps)
    b1r = b1.reshape(1, OUT_CH)
    b2r = b2.reshape(1, OUT_CH)
    g0 = _make_sc_gather(SL0_CHUNKS, CPW0)(a_tab, b_tab, idx0[0], idx0[1])
    g1 = _make_sc_gather(SL1_CHUNKS, CPW1)(a_tab, b_tab, idx1[0], idx1[1])
    acc = _mlp_part(None, g0, b1r, W2, b2r, 0)
    acc = _mlp_part(acc, g1, b1r, W2, b2r, SL0_CHUNKS * CHUNK // EDGE_BLK)
    return acc


# speedup vs baseline: 1.0125x; 1.0125x over previous
import functools

import jax
import jax.numpy as jnp
from jax import lax
from jax.experimental import pallas as pl
from jax.experimental.pallas import tpu as pltpu
from jax.experimental.pallas import tpu_sc as plsc

IN_CH = 128
OUT_CH = 128
N_NODES = 10000
N_EDGES = 320000

CHUNK = 128
N_CHUNKS = N_EDGES // CHUNK
NC = 2
NS = 16
NW = NC * NS
CH_PER_W = 80
PAD_CHUNKS = CH_PER_W * NW
NBUF = 5
DA = 2
DW = 4

NODE_BLK = 2000


def _tables_body(x_ref, w1_ref, eps_ref, a_ref, b_ref):
    xw = jnp.dot(x_ref[...], w1_ref[...], preferred_element_type=jnp.float32)
    b_ref[...] = xw
    a_ref[...] = (1.0 + eps_ref[0]) * xw


def _make_tables(x, W1, eps):
    grid = (N_NODES // NODE_BLK,)
    return pl.pallas_call(
        _tables_body,
        grid=grid,
        in_specs=[
            pl.BlockSpec((NODE_BLK, IN_CH), lambda i: (i, 0)),
            pl.BlockSpec((IN_CH, OUT_CH), lambda i: (0, 0)),
            pl.BlockSpec(memory_space=pltpu.SMEM),
        ],
        out_specs=[
            pl.BlockSpec((NODE_BLK, OUT_CH), lambda i: (i, 0)),
            pl.BlockSpec((NODE_BLK, OUT_CH), lambda i: (i, 0)),
        ],
        out_shape=[
            jax.ShapeDtypeStruct((N_NODES, OUT_CH), jnp.float32),
            jax.ShapeDtypeStruct((N_NODES, OUT_CH), jnp.float32),
        ],
    )(x, W1, eps)


def _sc_gather_body(a_hbm, b_hbm, row_hbm, col_hbm, g_hbm,
                    idx_r, idx_c, rows0, rows1, rows2, rows3, rows4,
                    sa0, sa1, sa2, sa3, sa4, sb0, sb1, sb2, sb3, sb4):
    wid = lax.axis_index("s") * NC + lax.axis_index("c")
    base_chunk = wid * CH_PER_W
    rows = (rows0, rows1, rows2, rows3, rows4)
    sem_a = (sa0, sa1, sa2, sa3, sa4)
    sem_b = (sb0, sb1, sb2, sb3, sb4)

    pltpu.sync_copy(row_hbm.at[pl.ds(base_chunk, CH_PER_W)], idx_r)
    pltpu.sync_copy(col_hbm.at[pl.ds(base_chunk, CH_PER_W)], idx_c)

    def valid(k):
        return (k >= 0) & (k < CH_PER_W) & (base_chunk + k < N_CHUNKS)

    @pl.loop(0, CH_PER_W + NBUF, step=NBUF)
    def _(k0):
        for b in range(NBUF):
            k = k0 + b

            @pl.when(valid(k))
            def _():
                pltpu.async_copy(a_hbm.at[idx_r.at[k]], rows[b], sem_a[b])

            ba = (b - DA) % NBUF
            ka = k - DA

            @pl.when(valid(ka))
            def _():
                pltpu.make_async_copy(
                    a_hbm.at[idx_r.at[ka]], rows[ba], sem_a[ba]
                ).wait()
                pltpu.async_copy(
                    b_hbm.at[idx_c.at[ka]], rows[ba], sem_b[ba],
                    add=True,
                )

            bw = (b - DW) % NBUF
            kw = k - DW

            @pl.when(valid(kw))
            def _():
                pltpu.make_async_copy(
                    b_hbm.at[idx_c.at[kw]], rows[bw], sem_b[bw]
                ).wait()
                pltpu.sync_copy(
                    rows[bw],
                    g_hbm.at[pl.ds((base_chunk + kw) * CHUNK, CHUNK)],
                )


@functools.cache
def _make_sc_gather():
    mesh = plsc.VectorSubcoreMesh(core_axis_name="c", subcore_axis_name="s")
    return pl.kernel(
        _sc_gather_body,
        out_type=jax.ShapeDtypeStruct((N_EDGES, OUT_CH), jnp.float32),
        mesh=mesh,
        scratch_types=[
            pltpu.VMEM((CH_PER_W, CHUNK), jnp.int32),
            pltpu.VMEM((CH_PER_W, CHUNK), jnp.int32),
            pltpu.VMEM((CHUNK, OUT_CH), jnp.float32),
            pltpu.VMEM((CHUNK, OUT_CH), jnp.float32),
            pltpu.VMEM((CHUNK, OUT_CH), jnp.float32),
            pltpu.VMEM((CHUNK, OUT_CH), jnp.float32),
            pltpu.VMEM((CHUNK, OUT_CH), jnp.float32),
            pltpu.SemaphoreType.DMA,
            pltpu.SemaphoreType.DMA,
            pltpu.SemaphoreType.DMA,
            pltpu.SemaphoreType.DMA,
            pltpu.SemaphoreType.DMA,
            pltpu.SemaphoreType.DMA,
            pltpu.SemaphoreType.DMA,
            pltpu.SemaphoreType.DMA,
            pltpu.SemaphoreType.DMA,
            pltpu.SemaphoreType.DMA,
        ],
    )


EDGE_BLK = 8000


def _mlp_body(g_ref, b1_ref, w2_ref, b2_ref, out_ref):
    h = jnp.maximum(g_ref[...] + b1_ref[...], 0.0)
    out_ref[...] = (
        jnp.dot(h, w2_ref[...], preferred_element_type=jnp.float32)
        + b2_ref[...]
    )


def _mlp(g, b1, W2, b2):
    grid = (N_EDGES // EDGE_BLK,)
    return pl.pallas_call(
        _mlp_body,
        grid=grid,
        in_specs=[
            pl.BlockSpec((EDGE_BLK, OUT_CH), lambda i: (i, 0)),
            pl.BlockSpec((1, OUT_CH), lambda i: (0, 0)),
            pl.BlockSpec((OUT_CH, OUT_CH), lambda i: (0, 0)),
            pl.BlockSpec((1, OUT_CH), lambda i: (0, 0)),
        ],
        out_specs=pl.BlockSpec((EDGE_BLK, OUT_CH), lambda i: (i, 0)),
        out_shape=jax.ShapeDtypeStruct((N_EDGES, OUT_CH), jnp.float32),
    )(g, b1.reshape(1, OUT_CH), W2, b2.reshape(1, OUT_CH))


def kernel(x, edge_index, W1, b1, W2, b2, eps):
    idx2d = edge_index.astype(jnp.int32).reshape(2, N_CHUNKS, CHUNK)
    pad = ((0, 0), (0, PAD_CHUNKS - N_CHUNKS), (0, 0))
    idx2d = jnp.pad(idx2d, pad)
    a_tab, b_tab = _make_tables(x, W1, eps)
    g = _make_sc_gather()(a_tab, b_tab, idx2d[0], idx2d[1])
    return _mlp(g, b1, W2, b2)
